# Initial kernel scaffold; baseline (speedup 1.0000x reference)
import jax
import jax.numpy as jnp
from jax.experimental import pallas as pl


def kernel(x, edge_index, edge_weight, Wp1, bp1, Wn1, bn1, Wp2, bp2, Wn2, bn2):
    raise NotImplementedError("write your pallas kernel here")


# speedup vs baseline: 9.1545x; 9.1545x over previous
import functools

import jax
import jax.numpy as jnp
from jax import lax
from jax.experimental import pallas as pl
from jax.experimental.pallas import tpu as pltpu
from jax.experimental.pallas import tpu_sc as plsc

NC = 2
NS = 16
NW = NC * NS
L = 16

_SC_PARAMS = pltpu.CompilerParams(needs_layout_passes=False,
                                  use_tc_tiling_on_sc=False)


def _mesh():
    return plsc.VectorSubcoreMesh(
        core_axis_name="c", subcore_axis_name="s",
        num_cores=NC, num_subcores=NS)


def _wid():
    return lax.axis_index("s") * NC + lax.axis_index("c")


def _zero_ref(ref, nwords):
    z = jnp.zeros((L,), jnp.float32)

    def body(i, carry):
        ref[pl.ds(i * L, L)] = z
        return carry

    lax.fori_loop(0, nwords // L, body, 0)


@functools.cache
def _make_deg(N, E, CH):
    EPT = E // NW

    @functools.partial(
        pl.kernel,
        out_type=jax.ShapeDtypeStruct((NW, 2 * N), jnp.float32),
        mesh=_mesh(),
        scratch_types=[
            pltpu.VMEM((2 * N,), jnp.float32),
            pltpu.VMEM((CH,), jnp.int32),
            pltpu.VMEM((CH,), jnp.float32),
        ],
        compiler_params=_SC_PARAMS,
    )
    def deg_kernel(dst_hbm, w_hbm, out_hbm, acc, dbuf, wbuf):
        w = _wid()
        _zero_ref(acc, 2 * N)
        base = w * EPT
        for c in range(EPT // CH):
            off = base + c * CH
            pltpu.sync_copy(dst_hbm.at[pl.ds(off, CH)], dbuf)
            pltpu.sync_copy(w_hbm.at[pl.ds(off, CH)], wbuf)

            def body(i, carry):
                d = dbuf[pl.ds(i * L, L)]
                ew = wbuf[pl.ds(i * L, L)]
                neg = (ew < 0.0).astype(jnp.int32)
                plsc.addupdate_scatter(acc, [d + neg * N], jnp.abs(ew))
                return carry

            lax.fori_loop(0, CH // L, body, 0)
        pltpu.sync_copy(acc, out_hbm.at[w])

    return deg_kernel


@functools.cache
def _make_norm(N, E, CH, F):
    EPT = E // NW

    @functools.partial(
        pl.kernel,
        out_type=(jax.ShapeDtypeStruct((E,), jnp.float32),
                  jax.ShapeDtypeStruct((E,), jnp.int32)),
        mesh=_mesh(),
        scratch_types=[
            pltpu.VMEM((2 * N,), jnp.float32),
            pltpu.VMEM((CH,), jnp.int32),
            pltpu.VMEM((CH,), jnp.int32),
            pltpu.VMEM((CH,), jnp.float32),
            pltpu.VMEM((CH,), jnp.float32),
            pltpu.VMEM((CH,), jnp.int32),
        ],
        compiler_params=_SC_PARAMS,
    )
    def norm_kernel(src_hbm, dst_hbm, w_hbm, dinv_hbm, norm_hbm, sd_hbm,
                    dinv, sbuf, dbuf, wbuf, nbuf, sdbuf):
        w = _wid()
        pltpu.sync_copy(dinv_hbm, dinv)
        base = w * EPT
        for c in range(EPT // CH):
            off = base + c * CH
            pltpu.sync_copy(src_hbm.at[pl.ds(off, CH)], sbuf)
            pltpu.sync_copy(dst_hbm.at[pl.ds(off, CH)], dbuf)
            pltpu.sync_copy(w_hbm.at[pl.ds(off, CH)], wbuf)

            def body(i, carry):
                s = sbuf[pl.ds(i * L, L)]
                d = dbuf[pl.ds(i * L, L)]
                ew = wbuf[pl.ds(i * L, L)]
                neg = (ew < 0.0).astype(jnp.int32)
                o = neg * N
                g1 = plsc.load_gather(dinv, [s + o])
                g2 = plsc.load_gather(dinv, [d + o])
                nbuf[pl.ds(i * L, L)] = g1 * g2 * jnp.abs(ew)
                sd = d + neg * (F * N)
                sdbuf[pl.ds(i * L, L)] = sd * 16384 + s
                return carry

            lax.fori_loop(0, CH // L, body, 0)
            pltpu.sync_copy(nbuf, norm_hbm.at[pl.ds(off, CH)])
            pltpu.sync_copy(sdbuf, sd_hbm.at[pl.ds(off, CH)])

    return norm_kernel


@functools.cache
def _make_scatter(N, E, CH, F):
    FN = F * N

    @functools.partial(
        pl.kernel,
        out_type=jax.ShapeDtypeStruct((2, NW, FN), jnp.float32),
        mesh=_mesh(),
        scratch_types=[
            pltpu.VMEM((FN,), jnp.float32),
            pltpu.VMEM((2 * FN,), jnp.float32),
            pltpu.VMEM((CH,), jnp.int32),
            pltpu.VMEM((CH,), jnp.float32),
        ],
        compiler_params=_SC_PARAMS,
    )
    def scat_kernel(u_hbm, sd_hbm, norm_hbm, out_hbm,
                    u, acc, sdbuf, nbuf):
        w = _wid()
        pltpu.sync_copy(u_hbm.at[w], u)
        _zero_ref(acc, 2 * FN)

        def chunk(c, carry):
            off = c * CH
            pltpu.sync_copy(sd_hbm.at[pl.ds(off, CH)], sdbuf)
            pltpu.sync_copy(norm_hbm.at[pl.ds(off, CH)], nbuf)

            def body(i, carry2):
                p = sdbuf[pl.ds(i * L, L)]
                s = lax.bitwise_and(p, 16383)
                sd = lax.shift_right_logical(p, 14)
                nm = nbuf[pl.ds(i * L, L)]
                for f in range(F):
                    v = plsc.load_gather(u, [s + f * N])
                    plsc.addupdate_scatter(acc, [sd + f * N], v * nm)
                return carry2

            lax.fori_loop(0, CH // L, body, 0)
            return carry

        lax.fori_loop(0, E // CH, chunk, 0)
        pltpu.sync_copy(acc.at[pl.ds(0, FN)], out_hbm.at[0, w])
        pltpu.sync_copy(acc.at[pl.ds(FN, FN)], out_hbm.at[1, w])

    return scat_kernel


def _tc_prep(degpart, N):
    def body(dp_ref, dinv_ref, dinv2_ref):
        deg = 1.0 + jnp.sum(dp_ref[...], axis=0, keepdims=True)
        dinv_ref[...] = lax.rsqrt(deg)
        dinv2_ref[...] = 1.0 / deg

    return pl.pallas_call(
        body,
        out_shape=(jax.ShapeDtypeStruct((1, 2 * N), jnp.float32),
                   jax.ShapeDtypeStruct((1, 2 * N), jnp.float32)),
    )(degpart)


def _tc_layer(S, uT, dinv2, WpT, WnT, bp, bn):
    D, Nn = uT.shape

    def body(s_ref, u_ref, d2_ref, wp_ref, wn_ref, bp_ref, bn_ref, o_ref):
        ap = s_ref[0] + u_ref[...] * d2_ref[0:1, :]
        an = s_ref[1] + u_ref[...] * d2_ref[1:2, :]
        hp = jnp.dot(wp_ref[...], ap, preferred_element_type=jnp.float32)
        hn = jnp.dot(wn_ref[...], an, preferred_element_type=jnp.float32)
        o_ref[...] = jnp.maximum(hp + bp_ref[...] - hn - bn_ref[...], 0.0)

    return pl.pallas_call(
        body,
        out_shape=jax.ShapeDtypeStruct((D, Nn), jnp.float32),
    )(S, uT, dinv2, WpT, WnT, bp, bn)


def kernel(x, edge_index, edge_weight, Wp1, bp1, Wn1, bn1, Wp2, bp2, Wn2, bn2):
    N, D = x.shape
    E = edge_weight.shape[0]
    F = D // NW
    FN = F * N
    CH = 2000
    assert E % (NW * CH) == 0 and D % NW == 0

    src = edge_index[0]
    dst = edge_index[1]

    degpart = _make_deg(N, E, CH)(dst, edge_weight)
    dinv, dinv2 = _tc_prep(degpart, N)
    norm, sd = _make_norm(N, E, CH, F)(src, dst, edge_weight,
                                       dinv.reshape(2 * N))
    d2 = dinv2.reshape(2, N)

    xT = x.T
    S1 = _make_scatter(N, E, CH, F)(xT.reshape(NW, FN), sd, norm)
    hT = _tc_layer(S1.reshape(2, D, N), xT, d2, Wp1.T, Wn1.T,
                   bp1.reshape(D, 1), bn1.reshape(D, 1))
    S2 = _make_scatter(N, E, CH, F)(hT.reshape(NW, FN), sd, norm)
    outT = _tc_layer(S2.reshape(2, D, N), hT, d2, Wp2.T, Wn2.T,
                     bp2.reshape(D, 1), bn2.reshape(D, 1))
    return outT.T


# speedup vs baseline: 12.1269x; 1.3247x over previous
import functools

import jax
import jax.numpy as jnp
from jax import lax
from jax.experimental import pallas as pl
from jax.experimental.pallas import tpu as pltpu
from jax.experimental.pallas import tpu_sc as plsc

NC = 2
NS = 16
NW = NC * NS
L = 16

_SC_PARAMS = pltpu.CompilerParams(needs_layout_passes=False,
                                  use_tc_tiling_on_sc=False)


def _mesh():
    return plsc.VectorSubcoreMesh(
        core_axis_name="c", subcore_axis_name="s",
        num_cores=NC, num_subcores=NS)


def _wid():
    return lax.axis_index("s") * NC + lax.axis_index("c")


def _zero_ref(ref, nwords):
    z = jnp.zeros((L,), jnp.float32)

    def body(i, carry):
        ref[pl.ds(i * L, L)] = z
        return carry

    lax.fori_loop(0, nwords // L, body, 0)


@functools.cache
def _make_deg(N, E, CH):
    EPT = E // NW

    @functools.partial(
        pl.kernel,
        out_type=jax.ShapeDtypeStruct((NW, 2 * N), jnp.float32),
        mesh=_mesh(),
        scratch_types=[
            pltpu.VMEM((2 * N,), jnp.float32),
            pltpu.VMEM((CH,), jnp.int32),
            pltpu.VMEM((CH,), jnp.float32),
        ],
        compiler_params=_SC_PARAMS,
    )
    def deg_kernel(dst_hbm, w_hbm, out_hbm, acc, dbuf, wbuf):
        w = _wid()
        _zero_ref(acc, 2 * N)
        base = w * EPT
        for c in range(EPT // CH):
            off = base + c * CH
            pltpu.sync_copy(dst_hbm.at[pl.ds(off, CH)], dbuf)
            pltpu.sync_copy(w_hbm.at[pl.ds(off, CH)], wbuf)

            def body(i, carry):
                d = dbuf[pl.ds(i * L, L)]
                ew = wbuf[pl.ds(i * L, L)]
                neg = (ew < 0.0).astype(jnp.int32)
                plsc.addupdate_scatter(acc, [d + neg * N], jnp.abs(ew))
                return carry

            lax.fori_loop(0, CH // L, body, 0)
        pltpu.sync_copy(acc, out_hbm.at[w])

    return deg_kernel


@functools.cache
def _make_norm(N, E, CH, F):
    EPT = E // NW

    @functools.partial(
        pl.kernel,
        out_type=(jax.ShapeDtypeStruct((E,), jnp.float32),
                  jax.ShapeDtypeStruct((E,), jnp.int32)),
        mesh=_mesh(),
        scratch_types=[
            pltpu.VMEM((2 * N,), jnp.float32),
            pltpu.VMEM((CH,), jnp.int32),
            pltpu.VMEM((CH,), jnp.int32),
            pltpu.VMEM((CH,), jnp.float32),
            pltpu.VMEM((CH,), jnp.float32),
            pltpu.VMEM((CH,), jnp.int32),
        ],
        compiler_params=_SC_PARAMS,
    )
    def norm_kernel(src_hbm, dst_hbm, w_hbm, dinv_hbm, norm_hbm, sd_hbm,
                    dinv, sbuf, dbuf, wbuf, nbuf, sdbuf):
        w = _wid()
        pltpu.sync_copy(dinv_hbm, dinv)
        base = w * EPT
        for c in range(EPT // CH):
            off = base + c * CH
            pltpu.sync_copy(src_hbm.at[pl.ds(off, CH)], sbuf)
            pltpu.sync_copy(dst_hbm.at[pl.ds(off, CH)], dbuf)
            pltpu.sync_copy(w_hbm.at[pl.ds(off, CH)], wbuf)

            def body(i, carry):
                s = sbuf[pl.ds(i * L, L)]
                d = dbuf[pl.ds(i * L, L)]
                ew = wbuf[pl.ds(i * L, L)]
                neg = (ew < 0.0).astype(jnp.int32)
                o = neg * N
                g1 = plsc.load_gather(dinv, [s + o])
                g2 = plsc.load_gather(dinv, [d + o])
                nbuf[pl.ds(i * L, L)] = g1 * g2 * jnp.abs(ew)
                sd = d + neg * (F * N)
                sdbuf[pl.ds(i * L, L)] = sd * 16384 + s
                return carry

            lax.fori_loop(0, CH // L, body, 0)
            pltpu.sync_copy(nbuf, norm_hbm.at[pl.ds(off, CH)])
            pltpu.sync_copy(sdbuf, sd_hbm.at[pl.ds(off, CH)])

    return norm_kernel


@functools.cache
def _make_scatter(N, E, CH, F):
    FN = F * N

    NCH = E // CH
    assert NCH % 2 == 0

    @functools.partial(
        pl.kernel,
        out_type=jax.ShapeDtypeStruct((2, NW, FN), jnp.float32),
        mesh=_mesh(),
        scratch_types=[
            pltpu.VMEM((FN,), jnp.float32),
            pltpu.VMEM((2 * FN,), jnp.float32),
            pltpu.VMEM((CH,), jnp.int32),
            pltpu.VMEM((CH,), jnp.int32),
            pltpu.VMEM((CH,), jnp.float32),
            pltpu.VMEM((CH,), jnp.float32),
            pltpu.SemaphoreType.DMA,
            pltpu.SemaphoreType.DMA,
        ],
        compiler_params=_SC_PARAMS,
    )
    def scat_kernel(u_hbm, sd_hbm, norm_hbm, out_hbm,
                    u, acc, sdb0, sdb1, nb0, nb1, sem0, sem1):
        w = _wid()
        pltpu.sync_copy(u_hbm.at[w], u)
        _zero_ref(acc, 2 * FN)

        sdb = (sdb0, sdb1)
        nbb = (nb0, nb1)
        sems = (sem0, sem1)
        for b in range(2):
            pltpu.async_copy(sd_hbm.at[pl.ds(b * CH, CH)], sdb[b], sems[b])
            pltpu.async_copy(norm_hbm.at[pl.ds(b * CH, CH)], nbb[b], sems[b])

        @pl.loop(0, NCH, step=2)
        def chunk(c0):
            for b in range(2):
                c = c0 + b
                off = c * CH
                pltpu.make_async_copy(
                    sd_hbm.at[pl.ds(off, CH)], sdb[b], sems[b]).wait()
                pltpu.make_async_copy(
                    norm_hbm.at[pl.ds(off, CH)], nbb[b], sems[b]).wait()
                sbuf, nbuf = sdb[b], nbb[b]

                def body(i, carry2):
                    p = sbuf[pl.ds(i * L, L)]
                    s = lax.bitwise_and(p, 16383)
                    sd = lax.shift_right_logical(p, 14)
                    nm = nbuf[pl.ds(i * L, L)]
                    for f in range(F):
                        v = plsc.load_gather(u, [s + f * N])
                        plsc.addupdate_scatter(acc, [sd + f * N], v * nm)
                    return carry2

                lax.fori_loop(0, CH // L, body, 0, unroll=4)

                @pl.when(c + 2 < NCH)
                def _():
                    nxt = off + 2 * CH
                    pltpu.async_copy(
                        sd_hbm.at[pl.ds(nxt, CH)], sdb[b], sems[b])
                    pltpu.async_copy(
                        norm_hbm.at[pl.ds(nxt, CH)], nbb[b], sems[b])

        pltpu.sync_copy(acc.at[pl.ds(0, FN)], out_hbm.at[0, w])
        pltpu.sync_copy(acc.at[pl.ds(FN, FN)], out_hbm.at[1, w])

    return scat_kernel


def _tc_prep(degpart, N):
    def body(dp_ref, dinv_ref, dinv2_ref):
        deg = 1.0 + jnp.sum(dp_ref[...], axis=0, keepdims=True)
        dinv_ref[...] = lax.rsqrt(deg)
        dinv2_ref[...] = 1.0 / deg

    return pl.pallas_call(
        body,
        out_shape=(jax.ShapeDtypeStruct((1, 2 * N), jnp.float32),
                   jax.ShapeDtypeStruct((1, 2 * N), jnp.float32)),
    )(degpart)


def _tc_layer(S, uT, dinv2, WpT, WnT, bp, bn):
    D, Nn = uT.shape

    def body(s_ref, u_ref, d2_ref, wp_ref, wn_ref, bp_ref, bn_ref, o_ref):
        ap = s_ref[0] + u_ref[...] * d2_ref[0:1, :]
        an = s_ref[1] + u_ref[...] * d2_ref[1:2, :]
        hp = jnp.dot(wp_ref[...], ap, preferred_element_type=jnp.float32)
        hn = jnp.dot(wn_ref[...], an, preferred_element_type=jnp.float32)
        o_ref[...] = jnp.maximum(hp + bp_ref[...] - hn - bn_ref[...], 0.0)

    return pl.pallas_call(
        body,
        out_shape=jax.ShapeDtypeStruct((D, Nn), jnp.float32),
    )(S, uT, dinv2, WpT, WnT, bp, bn)


def kernel(x, edge_index, edge_weight, Wp1, bp1, Wn1, bn1, Wp2, bp2, Wn2, bn2):
    N, D = x.shape
    E = edge_weight.shape[0]
    F = D // NW
    FN = F * N
    CH = 2000
    assert E % (NW * CH) == 0 and D % NW == 0

    src = edge_index[0]
    dst = edge_index[1]

    degpart = _make_deg(N, E, CH)(dst, edge_weight)
    dinv, dinv2 = _tc_prep(degpart, N)
    norm, sd = _make_norm(N, E, CH, F)(src, dst, edge_weight,
                                       dinv.reshape(2 * N))
    d2 = dinv2.reshape(2, N)

    xT = x.T
    S1 = _make_scatter(N, E, CH, F)(xT.reshape(NW, FN), sd, norm)
    hT = _tc_layer(S1.reshape(2, D, N), xT, d2, Wp1.T, Wn1.T,
                   bp1.reshape(D, 1), bn1.reshape(D, 1))
    S2 = _make_scatter(N, E, CH, F)(hT.reshape(NW, FN), sd, norm)
    outT = _tc_layer(S2.reshape(2, D, N), hT, d2, Wp2.T, Wn2.T,
                     bp2.reshape(D, 1), bn2.reshape(D, 1))
    return outT.T


# speedup vs baseline: 29.2885x; 2.4152x over previous
import functools

import jax
import jax.numpy as jnp
from jax import lax
from jax.experimental import pallas as pl
from jax.experimental.pallas import tpu as pltpu
from jax.experimental.pallas import tpu_sc as plsc

NC = 2
NS = 16
NW = NC * NS
L = 16

_SC_PARAMS = pltpu.CompilerParams(needs_layout_passes=False,
                                  use_tc_tiling_on_sc=False)


def _mesh():
    return plsc.VectorSubcoreMesh(
        core_axis_name="c", subcore_axis_name="s",
        num_cores=NC, num_subcores=NS)


def _wid():
    return lax.axis_index("s") * NC + lax.axis_index("c")


def _zero_ref(ref, nwords):
    z = jnp.zeros((L,), jnp.float32)

    def body(i, carry):
        ref[pl.ds(i * L, L)] = z
        return carry

    lax.fori_loop(0, nwords // L, body, 0)


@functools.cache
def _make_deg(N, E, CH):
    EPT = E // NW

    @functools.partial(
        pl.kernel,
        out_type=jax.ShapeDtypeStruct((NW, 2 * N), jnp.float32),
        mesh=_mesh(),
        scratch_types=[
            pltpu.VMEM((2 * N,), jnp.float32),
            pltpu.VMEM((CH,), jnp.int32),
            pltpu.VMEM((CH,), jnp.float32),
        ],
        compiler_params=_SC_PARAMS,
    )
    def deg_kernel(dst_hbm, w_hbm, out_hbm, acc, dbuf, wbuf):
        w = _wid()
        _zero_ref(acc, 2 * N)
        base = w * EPT
        for c in range(EPT // CH):
            off = base + c * CH
            pltpu.sync_copy(dst_hbm.at[pl.ds(off, CH)], dbuf)
            pltpu.sync_copy(w_hbm.at[pl.ds(off, CH)], wbuf)

            def body(i, carry):
                d = dbuf[pl.ds(i * L, L)]
                ew = wbuf[pl.ds(i * L, L)]
                neg = (ew < 0.0).astype(jnp.int32)
                plsc.addupdate_scatter(acc, [d + neg * N], jnp.abs(ew))
                return carry

            lax.fori_loop(0, CH // L, body, 0)
        pltpu.sync_copy(acc, out_hbm.at[w])

    return deg_kernel


@functools.cache
def _make_norm(N, E, CH, F):
    EPT = E // NW

    @functools.partial(
        pl.kernel,
        out_type=(jax.ShapeDtypeStruct((E,), jnp.float32),
                  jax.ShapeDtypeStruct((E,), jnp.int32)),
        mesh=_mesh(),
        scratch_types=[
            pltpu.VMEM((2 * N,), jnp.float32),
            pltpu.VMEM((CH,), jnp.int32),
            pltpu.VMEM((CH,), jnp.int32),
            pltpu.VMEM((CH,), jnp.float32),
            pltpu.VMEM((CH,), jnp.float32),
            pltpu.VMEM((CH,), jnp.int32),
        ],
        compiler_params=_SC_PARAMS,
    )
    def norm_kernel(src_hbm, dst_hbm, w_hbm, dinv_hbm, norm_hbm, sd_hbm,
                    dinv, sbuf, dbuf, wbuf, nbuf, sdbuf):
        w = _wid()
        pltpu.sync_copy(dinv_hbm, dinv)
        base = w * EPT
        for c in range(EPT // CH):
            off = base + c * CH
            pltpu.sync_copy(src_hbm.at[pl.ds(off, CH)], sbuf)
            pltpu.sync_copy(dst_hbm.at[pl.ds(off, CH)], dbuf)
            pltpu.sync_copy(w_hbm.at[pl.ds(off, CH)], wbuf)

            def body(i, carry):
                s = sbuf[pl.ds(i * L, L)]
                d = dbuf[pl.ds(i * L, L)]
                ew = wbuf[pl.ds(i * L, L)]
                neg = (ew < 0.0).astype(jnp.int32)
                o = neg * N
                g1 = plsc.load_gather(dinv, [s + o])
                g2 = plsc.load_gather(dinv, [d + o])
                nbuf[pl.ds(i * L, L)] = g1 * g2 * jnp.abs(ew)
                sd = d + neg * (F * N)
                sdbuf[pl.ds(i * L, L)] = sd * 16384 + s
                return carry

            lax.fori_loop(0, CH // L, body, 0)
            pltpu.sync_copy(nbuf, norm_hbm.at[pl.ds(off, CH)])
            pltpu.sync_copy(sdbuf, sd_hbm.at[pl.ds(off, CH)])

    return norm_kernel


@functools.cache
def _make_scatter(N, E, CH, F):
    FN = F * N

    NCH = E // CH
    assert NCH % 2 == 0

    @functools.partial(
        pl.kernel,
        out_type=jax.ShapeDtypeStruct((2, NW, FN), jnp.float32),
        mesh=_mesh(),
        scratch_types=[
            pltpu.VMEM((FN,), jnp.float32),
            pltpu.VMEM((2 * FN,), jnp.float32),
            pltpu.VMEM((CH,), jnp.int32),
            pltpu.VMEM((CH,), jnp.int32),
            pltpu.VMEM((CH,), jnp.float32),
            pltpu.VMEM((CH,), jnp.float32),
            pltpu.SemaphoreType.DMA,
            pltpu.SemaphoreType.DMA,
        ],
        compiler_params=_SC_PARAMS,
    )
    def scat_kernel(u_hbm, sd_hbm, norm_hbm, out_hbm,
                    u, acc, sdb0, sdb1, nb0, nb1, sem0, sem1):
        w = _wid()
        pltpu.sync_copy(u_hbm.at[w], u)
        _zero_ref(acc, 2 * FN)

        sdb = (sdb0, sdb1)
        nbb = (nb0, nb1)
        sems = (sem0, sem1)
        for b in range(2):
            pltpu.async_copy(sd_hbm.at[pl.ds(b * CH, CH)], sdb[b], sems[b])
            pltpu.async_copy(norm_hbm.at[pl.ds(b * CH, CH)], nbb[b], sems[b])

        @pl.loop(0, NCH, step=2)
        def chunk(c0):
            for b in range(2):
                c = c0 + b
                off = c * CH
                pltpu.make_async_copy(
                    sd_hbm.at[pl.ds(off, CH)], sdb[b], sems[b]).wait()
                pltpu.make_async_copy(
                    norm_hbm.at[pl.ds(off, CH)], nbb[b], sems[b]).wait()
                sbuf, nbuf = sdb[b], nbb[b]

                @plsc.parallel_loop(0, CH // L, unroll=4)
                def body(i):
                    p = sbuf[pl.ds(i * L, L)]
                    s = lax.bitwise_and(p, 16383)
                    sd = lax.shift_right_logical(p, 14)
                    nm = nbuf[pl.ds(i * L, L)]
                    for f in range(F):
                        v = plsc.load_gather(u, [s + f * N])
                        plsc.addupdate_scatter(acc, [sd + f * N], v * nm)

                @pl.when(c + 2 < NCH)
                def _():
                    nxt = off + 2 * CH
                    pltpu.async_copy(
                        sd_hbm.at[pl.ds(nxt, CH)], sdb[b], sems[b])
                    pltpu.async_copy(
                        norm_hbm.at[pl.ds(nxt, CH)], nbb[b], sems[b])

        pltpu.sync_copy(acc.at[pl.ds(0, FN)], out_hbm.at[0, w])
        pltpu.sync_copy(acc.at[pl.ds(FN, FN)], out_hbm.at[1, w])

    return scat_kernel


def _tc_prep(degpart, N):
    def body(dp_ref, dinv_ref, dinv2_ref):
        deg = 1.0 + jnp.sum(dp_ref[...], axis=0, keepdims=True)
        dinv_ref[...] = lax.rsqrt(deg)
        dinv2_ref[...] = 1.0 / deg

    return pl.pallas_call(
        body,
        out_shape=(jax.ShapeDtypeStruct((1, 2 * N), jnp.float32),
                   jax.ShapeDtypeStruct((1, 2 * N), jnp.float32)),
    )(degpart)


def _tc_layer(S, uT, dinv2, WpT, WnT, bp, bn):
    D, Nn = uT.shape

    def body(s_ref, u_ref, d2_ref, wp_ref, wn_ref, bp_ref, bn_ref, o_ref):
        ap = s_ref[0] + u_ref[...] * d2_ref[0:1, :]
        an = s_ref[1] + u_ref[...] * d2_ref[1:2, :]
        hp = jnp.dot(wp_ref[...], ap, preferred_element_type=jnp.float32)
        hn = jnp.dot(wn_ref[...], an, preferred_element_type=jnp.float32)
        o_ref[...] = jnp.maximum(hp + bp_ref[...] - hn - bn_ref[...], 0.0)

    return pl.pallas_call(
        body,
        out_shape=jax.ShapeDtypeStruct((D, Nn), jnp.float32),
    )(S, uT, dinv2, WpT, WnT, bp, bn)


def kernel(x, edge_index, edge_weight, Wp1, bp1, Wn1, bn1, Wp2, bp2, Wn2, bn2):
    N, D = x.shape
    E = edge_weight.shape[0]
    F = D // NW
    FN = F * N
    CH = 2000
    assert E % (NW * CH) == 0 and D % NW == 0

    src = edge_index[0]
    dst = edge_index[1]

    degpart = _make_deg(N, E, CH)(dst, edge_weight)
    dinv, dinv2 = _tc_prep(degpart, N)
    norm, sd = _make_norm(N, E, CH, F)(src, dst, edge_weight,
                                       dinv.reshape(2 * N))
    d2 = dinv2.reshape(2, N)

    xT = x.T
    S1 = _make_scatter(N, E, CH, F)(xT.reshape(NW, FN), sd, norm)
    hT = _tc_layer(S1.reshape(2, D, N), xT, d2, Wp1.T, Wn1.T,
                   bp1.reshape(D, 1), bn1.reshape(D, 1))
    S2 = _make_scatter(N, E, CH, F)(hT.reshape(NW, FN), sd, norm)
    outT = _tc_layer(S2.reshape(2, D, N), hT, d2, Wp2.T, Wn2.T,
                     bp2.reshape(D, 1), bn2.reshape(D, 1))
    return outT.T


# speedup vs baseline: 30.9654x; 1.0573x over previous
import functools

import jax
import jax.numpy as jnp
from jax import lax
from jax.experimental import pallas as pl
from jax.experimental.pallas import tpu as pltpu
from jax.experimental.pallas import tpu_sc as plsc

NC = 2
NS = 16
NW = NC * NS
L = 16

_SC_PARAMS = pltpu.CompilerParams(needs_layout_passes=False,
                                  use_tc_tiling_on_sc=False)


def _mesh():
    return plsc.VectorSubcoreMesh(
        core_axis_name="c", subcore_axis_name="s",
        num_cores=NC, num_subcores=NS)


def _wid():
    return lax.axis_index("s") * NC + lax.axis_index("c")


def _zero_ref(ref, nwords):
    z = jnp.zeros((L,), jnp.float32)

    @plsc.parallel_loop(0, nwords // L, unroll=8)
    def body(i):
        ref[pl.ds(i * L, L)] = z


@functools.cache
def _make_deg(N, E, CH):
    EPT = E // NW

    @functools.partial(
        pl.kernel,
        out_type=jax.ShapeDtypeStruct((NW, 2 * N), jnp.float32),
        mesh=_mesh(),
        scratch_types=[
            pltpu.VMEM((2 * N,), jnp.float32),
            pltpu.VMEM((CH,), jnp.int32),
            pltpu.VMEM((CH,), jnp.float32),
        ],
        compiler_params=_SC_PARAMS,
    )
    def deg_kernel(dst_hbm, w_hbm, out_hbm, acc, dbuf, wbuf):
        w = _wid()
        _zero_ref(acc, 2 * N)
        base = w * EPT
        for c in range(EPT // CH):
            off = base + c * CH
            pltpu.sync_copy(dst_hbm.at[pl.ds(off, CH)], dbuf)
            pltpu.sync_copy(w_hbm.at[pl.ds(off, CH)], wbuf)

            @plsc.parallel_loop(0, CH // L, unroll=4)
            def body(i):
                d = dbuf[pl.ds(i * L, L)]
                ew = wbuf[pl.ds(i * L, L)]
                neg = (ew < 0.0).astype(jnp.int32)
                plsc.addupdate_scatter(acc, [d + neg * N], jnp.abs(ew))

        pltpu.sync_copy(acc, out_hbm.at[w])

    return deg_kernel


@functools.cache
def _make_norm(N, E, CH, F):
    EPT = E // NW

    @functools.partial(
        pl.kernel,
        out_type=(jax.ShapeDtypeStruct((E,), jnp.float32),
                  jax.ShapeDtypeStruct((E,), jnp.int32)),
        mesh=_mesh(),
        scratch_types=[
            pltpu.VMEM((2 * N,), jnp.float32),
            pltpu.VMEM((CH,), jnp.int32),
            pltpu.VMEM((CH,), jnp.int32),
            pltpu.VMEM((CH,), jnp.float32),
            pltpu.VMEM((CH,), jnp.float32),
            pltpu.VMEM((CH,), jnp.int32),
        ],
        compiler_params=_SC_PARAMS,
    )
    def norm_kernel(src_hbm, dst_hbm, w_hbm, dinv_hbm, norm_hbm, sd_hbm,
                    dinv, sbuf, dbuf, wbuf, nbuf, sdbuf):
        w = _wid()
        pltpu.sync_copy(dinv_hbm, dinv)
        base = w * EPT
        for c in range(EPT // CH):
            off = base + c * CH
            pltpu.sync_copy(src_hbm.at[pl.ds(off, CH)], sbuf)
            pltpu.sync_copy(dst_hbm.at[pl.ds(off, CH)], dbuf)
            pltpu.sync_copy(w_hbm.at[pl.ds(off, CH)], wbuf)

            @plsc.parallel_loop(0, CH // L, unroll=4)
            def body(i):
                s = sbuf[pl.ds(i * L, L)]
                d = dbuf[pl.ds(i * L, L)]
                ew = wbuf[pl.ds(i * L, L)]
                neg = (ew < 0.0).astype(jnp.int32)
                o = neg * N
                g1 = plsc.load_gather(dinv, [s + o])
                g2 = plsc.load_gather(dinv, [d + o])
                nbuf[pl.ds(i * L, L)] = g1 * g2 * jnp.abs(ew)
                sd = d + neg * (F * N)
                sdbuf[pl.ds(i * L, L)] = sd * 16384 + s

            pltpu.sync_copy(nbuf, norm_hbm.at[pl.ds(off, CH)])
            pltpu.sync_copy(sdbuf, sd_hbm.at[pl.ds(off, CH)])

    return norm_kernel


@functools.cache
def _make_scatter(N, E, CH, F):
    FN = F * N

    NCH = E // CH
    assert NCH % 2 == 0

    @functools.partial(
        pl.kernel,
        out_type=jax.ShapeDtypeStruct((2, NW, FN), jnp.float32),
        mesh=_mesh(),
        scratch_types=[
            pltpu.VMEM((FN // 2,), jnp.int32),
            pltpu.VMEM((2 * FN,), jnp.float32),
            pltpu.VMEM((CH,), jnp.int32),
            pltpu.VMEM((CH,), jnp.int32),
            pltpu.VMEM((CH,), jnp.float32),
            pltpu.VMEM((CH,), jnp.float32),
            pltpu.SemaphoreType.DMA,
            pltpu.SemaphoreType.DMA,
            pltpu.SemaphoreType.DMA,
        ],
        compiler_params=_SC_PARAMS,
    )
    def scat_kernel(u_hbm, sd_hbm, norm_hbm, out_hbm,
                    u, acc, sdb0, sdb1, nb0, nb1, sem0, sem1, usem):
        w = _wid()
        ucopy = pltpu.async_copy(u_hbm.at[w], u, usem)
        _zero_ref(acc, 2 * FN)
        ucopy.wait()

        sdb = (sdb0, sdb1)
        nbb = (nb0, nb1)
        sems = (sem0, sem1)
        cbase = w * (NCH // NW)

        def coff(c):
            cr = cbase + c
            cr = jnp.where(cr >= NCH, cr - NCH, cr)
            return cr * CH

        for b in range(2):
            pltpu.async_copy(sd_hbm.at[pl.ds(coff(b), CH)], sdb[b], sems[b])
            pltpu.async_copy(norm_hbm.at[pl.ds(coff(b), CH)], nbb[b], sems[b])

        @pl.loop(0, NCH, step=2)
        def chunk(c0):
            for b in range(2):
                c = c0 + b
                off = coff(c)
                pltpu.make_async_copy(
                    sd_hbm.at[pl.ds(off, CH)], sdb[b], sems[b]).wait()
                pltpu.make_async_copy(
                    norm_hbm.at[pl.ds(off, CH)], nbb[b], sems[b]).wait()
                sbuf, nbuf = sdb[b], nbb[b]

                @plsc.parallel_loop(0, CH // L, unroll=8)
                def body(i):
                    p = sbuf[pl.ds(i * L, L)]
                    s = lax.bitwise_and(p, 16383)
                    sd = lax.shift_right_logical(p, 14)
                    nm = nbuf[pl.ds(i * L, L)]
                    for q in range(F // 2):
                        v32 = plsc.load_gather(u, [s + q * N])
                        vlo = plsc.bitcast(
                            lax.shift_left(v32, 16), jnp.float32)
                        vhi = plsc.bitcast(
                            lax.bitwise_and(v32, jnp.int32(-65536)),
                            jnp.float32)
                        plsc.addupdate_scatter(
                            acc, [sd + (2 * q) * N], vlo * nm)
                        plsc.addupdate_scatter(
                            acc, [sd + (2 * q + 1) * N], vhi * nm)

                @pl.when(c + 2 < NCH)
                def _():
                    nxt = coff(c + 2)
                    pltpu.async_copy(
                        sd_hbm.at[pl.ds(nxt, CH)], sdb[b], sems[b])
                    pltpu.async_copy(
                        norm_hbm.at[pl.ds(nxt, CH)], nbb[b], sems[b])

        pltpu.sync_copy(acc.at[pl.ds(0, FN)], out_hbm.at[0, w])
        pltpu.sync_copy(acc.at[pl.ds(FN, FN)], out_hbm.at[1, w])

    return scat_kernel


def _tc_prep(degpart, N):
    def body(dp_ref, dinv_ref, dinv2_ref):
        deg = 1.0 + jnp.sum(dp_ref[...], axis=0, keepdims=True)
        dinv_ref[...] = lax.rsqrt(deg)
        dinv2_ref[...] = 1.0 / deg

    return pl.pallas_call(
        body,
        out_shape=(jax.ShapeDtypeStruct((1, 2 * N), jnp.float32),
                   jax.ShapeDtypeStruct((1, 2 * N), jnp.float32)),
    )(degpart)


def _tc_layer(S, uT, dinv2, WpT, WnT, bp, bn):
    D, Nn = uT.shape

    def body(s_ref, u_ref, d2_ref, wp_ref, wn_ref, bp_ref, bn_ref, o_ref):
        ap = s_ref[0] + u_ref[...] * d2_ref[0:1, :]
        an = s_ref[1] + u_ref[...] * d2_ref[1:2, :]
        hp = jnp.dot(wp_ref[...], ap, preferred_element_type=jnp.float32)
        hn = jnp.dot(wn_ref[...], an, preferred_element_type=jnp.float32)
        o_ref[...] = jnp.maximum(hp + bp_ref[...] - hn - bn_ref[...], 0.0)

    return pl.pallas_call(
        body,
        out_shape=jax.ShapeDtypeStruct((D, Nn), jnp.float32),
    )(S, uT, dinv2, WpT, WnT, bp, bn)


def _tc_layer_nm(S, uT, dinv2, Wp, Wn, bp, bn):
    D, Nn = uT.shape
    dn = (((0,), (0,)), ((), ()))

    def body(s_ref, u_ref, d2_ref, wp_ref, wn_ref, bp_ref, bn_ref, o_ref):
        ap = s_ref[0] + u_ref[...] * d2_ref[0:1, :]
        an = s_ref[1] + u_ref[...] * d2_ref[1:2, :]
        hp = lax.dot_general(ap, wp_ref[...], dn,
                             preferred_element_type=jnp.float32)
        hn = lax.dot_general(an, wn_ref[...], dn,
                             preferred_element_type=jnp.float32)
        o_ref[...] = jnp.maximum(hp + bp_ref[...] - hn - bn_ref[...], 0.0)

    return pl.pallas_call(
        body,
        out_shape=jax.ShapeDtypeStruct((Nn, D), jnp.float32),
    )(S, uT, dinv2, Wp, Wn, bp, bn)


def _pack_bf16_pairs(uT):
    D, Nn = uT.shape
    ub = uT.astype(jnp.bfloat16).reshape(NW, D // (2 * NW), 2, Nn)
    ub = ub.transpose(0, 1, 3, 2)
    return jax.lax.bitcast_convert_type(ub, jnp.int32).reshape(NW, -1)


def kernel(x, edge_index, edge_weight, Wp1, bp1, Wn1, bn1, Wp2, bp2, Wn2, bn2):
    N, D = x.shape
    E = edge_weight.shape[0]
    F = D // NW
    FN = F * N
    CH = 2000
    assert E % (NW * CH) == 0 and D % NW == 0

    src = edge_index[0]
    dst = edge_index[1]

    degpart = _make_deg(N, E, CH)(dst, edge_weight)
    dinv, dinv2 = _tc_prep(degpart, N)
    norm, sd = _make_norm(N, E, CH, F)(src, dst, edge_weight,
                                       dinv.reshape(2 * N))
    d2 = dinv2.reshape(2, N)

    xT = x.T
    S1 = _make_scatter(N, E, CH, F)(_pack_bf16_pairs(xT), sd, norm)
    hT = _tc_layer(S1.reshape(2, D, N), xT, d2, Wp1.T, Wn1.T,
                   bp1.reshape(D, 1), bn1.reshape(D, 1))
    S2 = _make_scatter(N, E, CH, F)(_pack_bf16_pairs(hT), sd, norm)
    return _tc_layer_nm(S2.reshape(2, D, N), hT, d2, Wp2, Wn2,
                        bp2.reshape(1, D), bn2.reshape(1, D))


# speedup vs baseline: 31.0691x; 1.0033x over previous
import functools

import jax
import jax.numpy as jnp
from jax import lax
from jax.experimental import pallas as pl
from jax.experimental.pallas import tpu as pltpu
from jax.experimental.pallas import tpu_sc as plsc

NC = 2
NS = 16
NW = NC * NS
L = 16

_SC_PARAMS = pltpu.CompilerParams(needs_layout_passes=False,
                                  use_tc_tiling_on_sc=False)


def _mesh():
    return plsc.VectorSubcoreMesh(
        core_axis_name="c", subcore_axis_name="s",
        num_cores=NC, num_subcores=NS)


def _wid():
    return lax.axis_index("s") * NC + lax.axis_index("c")


def _zero_ref(ref, nwords):
    z = jnp.zeros((L,), jnp.float32)

    @plsc.parallel_loop(0, nwords // L, unroll=8)
    def body(i):
        ref[pl.ds(i * L, L)] = z


@functools.cache
def _make_deg(N, E, CH):
    EPT = E // NW

    @functools.partial(
        pl.kernel,
        out_type=jax.ShapeDtypeStruct((NW, 2 * N), jnp.float32),
        mesh=_mesh(),
        scratch_types=[
            pltpu.VMEM((2 * N,), jnp.float32),
            pltpu.VMEM((CH,), jnp.int32),
            pltpu.VMEM((CH,), jnp.float32),
        ],
        compiler_params=_SC_PARAMS,
    )
    def deg_kernel(dst_hbm, w_hbm, out_hbm, acc, dbuf, wbuf):
        w = _wid()
        _zero_ref(acc, 2 * N)
        base = w * EPT
        for c in range(EPT // CH):
            off = base + c * CH
            pltpu.sync_copy(dst_hbm.at[pl.ds(off, CH)], dbuf)
            pltpu.sync_copy(w_hbm.at[pl.ds(off, CH)], wbuf)

            @plsc.parallel_loop(0, CH // L, unroll=4)
            def body(i):
                d = dbuf[pl.ds(i * L, L)]
                ew = wbuf[pl.ds(i * L, L)]
                neg = (ew < 0.0).astype(jnp.int32)
                plsc.addupdate_scatter(acc, [d + neg * N], jnp.abs(ew))

        pltpu.sync_copy(acc, out_hbm.at[w])

    return deg_kernel


@functools.cache
def _make_norm(N, E, CH, F):
    EPT = E // NW

    @functools.partial(
        pl.kernel,
        out_type=(jax.ShapeDtypeStruct((E,), jnp.float32),
                  jax.ShapeDtypeStruct((E,), jnp.int32)),
        mesh=_mesh(),
        scratch_types=[
            pltpu.VMEM((2 * N,), jnp.float32),
            pltpu.VMEM((CH,), jnp.int32),
            pltpu.VMEM((CH,), jnp.int32),
            pltpu.VMEM((CH,), jnp.float32),
            pltpu.VMEM((CH,), jnp.float32),
            pltpu.VMEM((CH,), jnp.int32),
        ],
        compiler_params=_SC_PARAMS,
    )
    def norm_kernel(src_hbm, dst_hbm, w_hbm, dinv_hbm, norm_hbm, sd_hbm,
                    dinv, sbuf, dbuf, wbuf, nbuf, sdbuf):
        w = _wid()
        pltpu.sync_copy(dinv_hbm, dinv)
        base = w * EPT
        for c in range(EPT // CH):
            off = base + c * CH
            pltpu.sync_copy(src_hbm.at[pl.ds(off, CH)], sbuf)
            pltpu.sync_copy(dst_hbm.at[pl.ds(off, CH)], dbuf)
            pltpu.sync_copy(w_hbm.at[pl.ds(off, CH)], wbuf)

            @plsc.parallel_loop(0, CH // L, unroll=4)
            def body(i):
                s = sbuf[pl.ds(i * L, L)]
                d = dbuf[pl.ds(i * L, L)]
                ew = wbuf[pl.ds(i * L, L)]
                neg = (ew < 0.0).astype(jnp.int32)
                o = neg * N
                g1 = plsc.load_gather(dinv, [s + o])
                g2 = plsc.load_gather(dinv, [d + o])
                nbuf[pl.ds(i * L, L)] = g1 * g2 * jnp.abs(ew)
                sd = d + neg * (F * N)
                sdbuf[pl.ds(i * L, L)] = sd * 16384 + s

            pltpu.sync_copy(nbuf, norm_hbm.at[pl.ds(off, CH)])
            pltpu.sync_copy(sdbuf, sd_hbm.at[pl.ds(off, CH)])

    return norm_kernel


@functools.cache
def _make_scatter(N, E, CH, F):
    FN = F * N

    NCH = E // CH
    assert NCH % 2 == 0

    @functools.partial(
        pl.kernel,
        out_type=jax.ShapeDtypeStruct((2, NW, FN), jnp.float32),
        mesh=_mesh(),
        scratch_types=[
            pltpu.VMEM((FN,), jnp.float32),
            pltpu.VMEM((2 * FN,), jnp.float32),
            pltpu.VMEM((CH,), jnp.int32),
            pltpu.VMEM((CH,), jnp.int32),
            pltpu.VMEM((CH,), jnp.float32),
            pltpu.VMEM((CH,), jnp.float32),
            pltpu.SemaphoreType.DMA,
            pltpu.SemaphoreType.DMA,
            pltpu.SemaphoreType.DMA,
        ],
        compiler_params=_SC_PARAMS,
    )
    def scat_kernel(u_hbm, sd_hbm, norm_hbm, out_hbm,
                    u, acc, sdb0, sdb1, nb0, nb1, sem0, sem1, usem):
        w = _wid()
        ucopy = pltpu.async_copy(u_hbm.at[w], u, usem)
        _zero_ref(acc, 2 * FN)
        ucopy.wait()

        sdb = (sdb0, sdb1)
        nbb = (nb0, nb1)
        sems = (sem0, sem1)
        cbase = w * (NCH // NW)

        def coff(c):
            cr = cbase + c
            cr = jnp.where(cr >= NCH, cr - NCH, cr)
            return cr * CH

        for b in range(2):
            pltpu.async_copy(sd_hbm.at[pl.ds(coff(b), CH)], sdb[b], sems[b])
            pltpu.async_copy(norm_hbm.at[pl.ds(coff(b), CH)], nbb[b], sems[b])

        @pl.loop(0, NCH, step=2)
        def chunk(c0):
            for b in range(2):
                c = c0 + b
                off = coff(c)
                pltpu.make_async_copy(
                    sd_hbm.at[pl.ds(off, CH)], sdb[b], sems[b]).wait()
                pltpu.make_async_copy(
                    norm_hbm.at[pl.ds(off, CH)], nbb[b], sems[b]).wait()
                sbuf, nbuf = sdb[b], nbb[b]

                @plsc.parallel_loop(0, CH // L, unroll=8)
                def body(i):
                    p = sbuf[pl.ds(i * L, L)]
                    s = lax.bitwise_and(p, 16383)
                    sd = lax.shift_right_logical(p, 14)
                    nm = nbuf[pl.ds(i * L, L)]
                    for f in range(F):
                        v = plsc.load_gather(u, [s + f * N])
                        plsc.addupdate_scatter(acc, [sd + f * N], v * nm)

                @pl.when(c + 2 < NCH)
                def _():
                    nxt = coff(c + 2)
                    pltpu.async_copy(
                        sd_hbm.at[pl.ds(nxt, CH)], sdb[b], sems[b])
                    pltpu.async_copy(
                        norm_hbm.at[pl.ds(nxt, CH)], nbb[b], sems[b])

        pltpu.sync_copy(acc.at[pl.ds(0, FN)], out_hbm.at[0, w])
        pltpu.sync_copy(acc.at[pl.ds(FN, FN)], out_hbm.at[1, w])

    return scat_kernel


def _tc_prep(degpart, N):
    def body(dp_ref, dinv_ref, dinv2_ref):
        deg = 1.0 + jnp.sum(dp_ref[...], axis=0, keepdims=True)
        dinv_ref[...] = lax.rsqrt(deg)
        dinv2_ref[...] = 1.0 / deg

    return pl.pallas_call(
        body,
        out_shape=(jax.ShapeDtypeStruct((1, 2 * N), jnp.float32),
                   jax.ShapeDtypeStruct((1, 2 * N), jnp.float32)),
    )(degpart)


def _tc_layer(S, uT, dinv2, WpT, WnT, bp, bn):
    D, Nn = uT.shape

    def body(s_ref, u_ref, d2_ref, wp_ref, wn_ref, bp_ref, bn_ref, o_ref):
        ap = s_ref[0] + u_ref[...] * d2_ref[0:1, :]
        an = s_ref[1] + u_ref[...] * d2_ref[1:2, :]
        hp = jnp.dot(wp_ref[...], ap, preferred_element_type=jnp.float32)
        hn = jnp.dot(wn_ref[...], an, preferred_element_type=jnp.float32)
        o_ref[...] = jnp.maximum(hp + bp_ref[...] - hn - bn_ref[...], 0.0)

    return pl.pallas_call(
        body,
        out_shape=jax.ShapeDtypeStruct((D, Nn), jnp.float32),
    )(S, uT, dinv2, WpT, WnT, bp, bn)


def _tc_layer_nm(S, uT, dinv2, Wp, Wn, bp, bn):
    D, Nn = uT.shape
    dn = (((0,), (0,)), ((), ()))

    def body(s_ref, u_ref, d2_ref, wp_ref, wn_ref, bp_ref, bn_ref, o_ref):
        ap = s_ref[0] + u_ref[...] * d2_ref[0:1, :]
        an = s_ref[1] + u_ref[...] * d2_ref[1:2, :]
        hp = lax.dot_general(ap, wp_ref[...], dn,
                             preferred_element_type=jnp.float32)
        hn = lax.dot_general(an, wn_ref[...], dn,
                             preferred_element_type=jnp.float32)
        o_ref[...] = jnp.maximum(hp + bp_ref[...] - hn - bn_ref[...], 0.0)

    return pl.pallas_call(
        body,
        out_shape=jax.ShapeDtypeStruct((Nn, D), jnp.float32),
    )(S, uT, dinv2, Wp, Wn, bp, bn)


def kernel(x, edge_index, edge_weight, Wp1, bp1, Wn1, bn1, Wp2, bp2, Wn2, bn2):
    N, D = x.shape
    E = edge_weight.shape[0]
    F = D // NW
    FN = F * N
    CH = 2000
    assert E % (NW * CH) == 0 and D % NW == 0

    src = edge_index[0]
    dst = edge_index[1]

    degpart = _make_deg(N, E, CH)(dst, edge_weight)
    dinv, dinv2 = _tc_prep(degpart, N)
    norm, sd = _make_norm(N, E, CH, F)(src, dst, edge_weight,
                                       dinv.reshape(2 * N))
    d2 = dinv2.reshape(2, N)

    xT = x.T
    S1 = _make_scatter(N, E, CH, F)(xT.reshape(NW, FN), sd, norm)
    hT = _tc_layer(S1.reshape(2, D, N), xT, d2, Wp1.T, Wn1.T,
                   bp1.reshape(D, 1), bn1.reshape(D, 1))
    S2 = _make_scatter(N, E, CH, F)(hT.reshape(NW, FN), sd, norm)
    return _tc_layer_nm(S2.reshape(2, D, N), hT, d2, Wp2, Wn2,
                        bp2.reshape(1, D), bn2.reshape(1, D))
